# baseline (speedup 1.0000x reference)
import jax
from jax.experimental import pallas as pl
from jax.experimental.pallas import tpu as pltpu


def _copy_body(src_ref, dst_ref, sem):
    cp = pltpu.make_async_copy(src_ref, dst_ref, sem)
    cp.start()
    cp.wait()


def kernel(data):
    return pl.pallas_call(
        _copy_body,
        out_shape=jax.ShapeDtypeStruct(data.shape, data.dtype),
        in_specs=[pl.BlockSpec(memory_space=pl.ANY)],
        out_specs=pl.BlockSpec(memory_space=pl.ANY),
        scratch_shapes=[pltpu.SemaphoreType.DMA],
    )(data)


# speedup vs baseline: 3.3372x; 3.3372x over previous
import jax
from jax.experimental import pallas as pl
from jax.experimental.pallas import tpu as pltpu

_ROWS = 28224
_LANES = 1024
_CHUNKS = 8
_CHUNK_ROWS = _ROWS // _CHUNKS


def _copy_body(src_ref, dst_ref, sems):
    copies = []
    for i in range(_CHUNKS):
        sl = pl.ds(i * _CHUNK_ROWS, _CHUNK_ROWS)
        copies.append(pltpu.make_async_copy(src_ref.at[sl], dst_ref.at[sl],
                                            sems.at[i]))
    for cp in copies:
        cp.start()
    for cp in copies:
        cp.wait()


def kernel(data):
    flat = data.reshape(_ROWS, _LANES)
    out = pl.pallas_call(
        _copy_body,
        out_shape=jax.ShapeDtypeStruct((_ROWS, _LANES), data.dtype),
        in_specs=[pl.BlockSpec(memory_space=pl.ANY)],
        out_specs=pl.BlockSpec(memory_space=pl.ANY),
        scratch_shapes=[pltpu.SemaphoreType.DMA((_CHUNKS,))],
    )(flat)
    return out.reshape(data.shape)


# speedup vs baseline: 10.7817x; 3.2308x over previous
import jax
from jax.experimental import pallas as pl
from jax.experimental.pallas import tpu as pltpu

_ROWS = 28224
_LANES = 1024
_GRID = 14
_BLOCK_ROWS = _ROWS // _GRID


def _copy_body(src_ref, dst_ref):
    dst_ref[...] = src_ref[...]


def kernel(data):
    flat = data.reshape(_ROWS, _LANES)
    out = pl.pallas_call(
        _copy_body,
        grid=(_GRID,),
        in_specs=[pl.BlockSpec((_BLOCK_ROWS, _LANES), lambda i: (i, 0))],
        out_specs=pl.BlockSpec((_BLOCK_ROWS, _LANES), lambda i: (i, 0)),
        out_shape=jax.ShapeDtypeStruct((_ROWS, _LANES), data.dtype),
    )(flat)
    return out.reshape(data.shape)


# speedup vs baseline: 230.6088x; 21.3889x over previous
import jax
import jax.numpy as jnp
from jax.experimental import pallas as pl

_ROWS = 6 * 28 * 28 * 16
_LANES = 384
_GRID = 16
_BLOCK_ROWS = _ROWS // _GRID


def _copy_body(src_ref, dst_ref):
    dst_ref[...] = src_ref[...]


def kernel(data):
    t, b, c, h, w = data.shape
    x = jnp.transpose(data, (0, 3, 4, 1, 2)).reshape(_ROWS, _LANES)
    out = pl.pallas_call(
        _copy_body,
        grid=(_GRID,),
        in_specs=[pl.BlockSpec((_BLOCK_ROWS, _LANES), lambda i: (i, 0))],
        out_specs=pl.BlockSpec((_BLOCK_ROWS, _LANES), lambda i: (i, 0)),
        out_shape=jax.ShapeDtypeStruct((_ROWS, _LANES), data.dtype),
    )(x)
    return jnp.transpose(out.reshape(t, h, w, b, c), (0, 3, 4, 1, 2))
